# baseline (device time: 177832 ns/iter reference)
import os
SKIP_COMM = bool(int(os.environ.get('SKIP_COMM', '0')))
import jax
import jax.numpy as jnp
from jax import lax
from jax.experimental import pallas as pl
from jax.experimental.pallas import tpu as pltpu

N_DEV = 32
B, Sq, Hq, Hkv, Dh = 4, 256, 8, 2, 128
G = Hq // Hkv
D = Hq * Dh
W = D + 128
R = B * Sq
CHUNK = R // N_DEV
SCALE = 0.08838834764831843
NR = N_DEV // 2
NL = N_DEV // 2 - 1
BF = jnp.bfloat16


def kernel(x, Wq, Wo, K_ext, V_ext):
    x_flat = x.reshape(R, D)

    def body(x_ref, wq_ref, wo_ref, k_ref, v_ref, out_ref,
             acc, lr_land, ll_land,
             sr_sem, sl_sem,
             r_sems, l_sems, ag_ssems, ag_rsems):
        my = lax.axis_index("i")
        left = lax.rem(my - 1 + N_DEV, N_DEV)
        right = lax.rem(my + 1, N_DEV)

        barrier = pltpu.get_barrier_semaphore()
        for nbr in (left, right):
            pl.semaphore_signal(
                barrier, inc=1,
                device_id=(nbr,), device_id_type=pl.DeviceIdType.MESH,
            )
        pl.semaphore_wait(barrier, 2)

        out_ref[:, :] = jnp.dot(
            x_ref[:, :].astype(BF), wq_ref[:, :].astype(BF),
            preferred_element_type=jnp.float32,
        )
        for b in range(B):
            rows = slice(b * Sq, (b + 1) * Sq)
            for g in range(Hkv):
                k16 = k_ref[b, :, g, :].astype(BF)
                v16 = v_ref[b, :, g, :].astype(BF)
                for h in range(g * G, (g + 1) * G):
                    q16 = out_ref[rows, h * Dh:(h + 1) * Dh].astype(BF)
                    s = lax.dot_general(
                        q16, k16, (((1,), (1,)), ((), ())),
                        preferred_element_type=jnp.float32,
                    ) * SCALE
                    p = jnp.exp(s)
                    l_loc = jnp.sum(p, axis=1, keepdims=True)
                    o = jnp.dot(p.astype(BF), v16,
                                preferred_element_type=jnp.float32)
                    acc[rows, h * Dh:(h + 1) * Dh] = o
                    acc[rows, D + h:D + h + 1] = l_loc

        def send_chunk(chunk, land, slot, ssem, rsems, dev):
            rows = pl.ds(chunk * CHUNK, CHUNK)
            rd = pltpu.make_async_remote_copy(
                src_ref=acc.at[rows, :],
                dst_ref=land.at[slot],
                send_sem=ssem, recv_sem=rsems.at[slot],
                device_id=(dev,), device_id_type=pl.DeviceIdType.MESH,
            )
            rd.start()
            return rd

        prev = []
        for s in range(0 if SKIP_COMM else NR):
            for rd in prev:
                rd.wait_send()
            sc_r = lax.rem(my + NR - s, N_DEV)
            pend = [send_chunk(sc_r, lr_land, s, sr_sem, r_sems, right)]
            if s < NL:
                sc_l = lax.rem(my - NL + s + N_DEV, N_DEV)
                pend.append(send_chunk(sc_l, ll_land, s, sl_sem, l_sems,
                                       left))
            for rd in pend:
                rd.wait_recv()
            prev = pend
            rc = lax.rem(my + NR - 1 - s + N_DEV, N_DEV)
            rows = pl.ds(rc * CHUNK, CHUNK)
            acc[rows, :] = acc[rows, :] + lr_land[s]
            if s < NL:
                rc = lax.rem(my - NL + 1 + s + N_DEV, N_DEV)
                rows = pl.ds(rc * CHUNK, CHUNK)
                acc[rows, :] = acc[rows, :] + ll_land[s]
        for rd in prev:
            rd.wait_send()

        orows = pl.ds(my * CHUNK, CHUNK)
        linv = 1.0 / acc[orows, D:D + Hq]
        och = acc[orows, 0:D]
        norm = jnp.concatenate(
            [och[:, h * Dh:(h + 1) * Dh] * linv[:, h:h + 1]
             for h in range(Hq)],
            axis=1,
        )
        out_ref[orows, :] = jnp.dot(
            norm.astype(BF), wo_ref[:, :].astype(BF),
            preferred_element_type=jnp.float32,
        )

        sends = []
        if not SKIP_COMM:
            for d in range(1, N_DEV):
                dest = lax.rem(my + d, N_DEV)
                rd = pltpu.make_async_remote_copy(
                    src_ref=out_ref.at[orows, :],
                    dst_ref=out_ref.at[orows, :],
                    send_sem=ag_ssems.at[d],
                    recv_sem=ag_rsems.at[N_DEV - d],
                    device_id=(dest,), device_id_type=pl.DeviceIdType.MESH,
                )
                rd.start()
                sends.append(rd)
            for e in range(1, N_DEV):
                rc = lax.rem(my + e, N_DEV)
                rrows = pl.ds(rc * CHUNK, CHUNK)
                rd = pltpu.make_async_remote_copy(
                    src_ref=out_ref.at[rrows, :],
                    dst_ref=out_ref.at[rrows, :],
                    send_sem=ag_ssems.at[e],
                    recv_sem=ag_rsems.at[e],
                    device_id=(right,), device_id_type=pl.DeviceIdType.MESH,
                )
                rd.wait_recv()
            for rd in sends:
                rd.wait_send()

    out = pl.pallas_call(
        body,
        out_shape=jax.ShapeDtypeStruct((R, D), jnp.float32),
        in_specs=[pl.BlockSpec(memory_space=pltpu.VMEM)] * 5,
        out_specs=pl.BlockSpec(memory_space=pltpu.VMEM),
        scratch_shapes=[
            pltpu.VMEM((R, W), jnp.float32),
            pltpu.VMEM((NR, CHUNK, W), jnp.float32),
            pltpu.VMEM((NL, CHUNK, W), jnp.float32),
            pltpu.SemaphoreType.DMA,
            pltpu.SemaphoreType.DMA,
            pltpu.SemaphoreType.DMA((NR,)),
            pltpu.SemaphoreType.DMA((NL,)),
            pltpu.SemaphoreType.DMA((N_DEV,)),
            pltpu.SemaphoreType.DMA((N_DEV,)),
        ],
        compiler_params=pltpu.CompilerParams(
            collective_id=0,
            vmem_limit_bytes=100 * 1024 * 1024,
        ),
    )(x_flat, Wq, Wo, K_ext, V_ext)
    return out.reshape(B, Sq, D)


# device time: 154328 ns/iter; 1.1523x vs baseline; 1.1523x over previous
import os
SKIP_COMM = bool(int(os.environ.get('SKIP_COMM', '0')))
import jax
import jax.numpy as jnp
from jax import lax
from jax.experimental import pallas as pl
from jax.experimental.pallas import tpu as pltpu

N_DEV = 32
B, Sq, Hq, Hkv, Dh = 4, 256, 8, 2, 128
G = Hq // Hkv
D = Hq * Dh
W = D + 128
R = B * Sq
CHUNK = R // N_DEV
SCALE = 0.08838834764831843
NR = N_DEV // 2
NL = N_DEV // 2 - 1
BF = jnp.bfloat16


def kernel(x, Wq, Wo, K_ext, V_ext):
    x_flat = x.reshape(R, D)

    def body(x_ref, wq_ref, wo_ref, k_ref, v_ref, out_ref,
             acc, rs_land,
             rs_ssems, rs_rsems, ag_ssems, ag_rsems):
        my = lax.axis_index("i")
        left = lax.rem(my - 1 + N_DEV, N_DEV)
        right = lax.rem(my + 1, N_DEV)

        barrier = pltpu.get_barrier_semaphore()
        for nbr in (left, right):
            pl.semaphore_signal(
                barrier, inc=1,
                device_id=(nbr,), device_id_type=pl.DeviceIdType.MESH,
            )
        pl.semaphore_wait(barrier, 2)

        out_ref[:, :] = jnp.dot(
            x_ref[:, :].astype(BF), wq_ref[:, :].astype(BF),
            preferred_element_type=jnp.float32,
        )
        for b in range(B):
            rows = slice(b * Sq, (b + 1) * Sq)
            for g in range(Hkv):
                k16 = k_ref[b, :, g, :].astype(BF)
                v16 = v_ref[b, :, g, :].astype(BF)
                for h in range(g * G, (g + 1) * G):
                    q16 = out_ref[rows, h * Dh:(h + 1) * Dh].astype(BF)
                    s = lax.dot_general(
                        q16, k16, (((1,), (1,)), ((), ())),
                        preferred_element_type=jnp.float32,
                    ) * SCALE
                    p = jnp.exp(s)
                    l_loc = jnp.sum(p, axis=1, keepdims=True)
                    o = jnp.dot(p.astype(BF), v16,
                                preferred_element_type=jnp.float32)
                    acc[rows, h * Dh:(h + 1) * Dh] = o
                    acc[rows, D + h:D + h + 1] = l_loc

        orows = pl.ds(my * CHUNK, CHUNK)
        rs_sends = []
        if not SKIP_COMM:
            for d in range(1, N_DEV):
                dest = lax.rem(my + d, N_DEV)
                drows = pl.ds(dest * CHUNK, CHUNK)
                rd = pltpu.make_async_remote_copy(
                    src_ref=acc.at[drows, :],
                    dst_ref=rs_land.at[N_DEV - d],
                    send_sem=rs_ssems.at[d],
                    recv_sem=rs_rsems.at[N_DEV - d],
                    device_id=(dest,), device_id_type=pl.DeviceIdType.MESH,
                )
                rd.start()
                rs_sends.append(rd)
            for e in range(1, N_DEV):
                rd = pltpu.make_async_remote_copy(
                    src_ref=acc.at[orows, :],
                    dst_ref=rs_land.at[e],
                    send_sem=rs_ssems.at[e],
                    recv_sem=rs_rsems.at[e],
                    device_id=(right,), device_id_type=pl.DeviceIdType.MESH,
                )
                rd.wait_recv()
                acc[orows, :] = acc[orows, :] + rs_land[e]
            for rd in rs_sends:
                rd.wait_send()

        linv = 1.0 / acc[orows, D:D + Hq]
        och = acc[orows, 0:D]
        norm = jnp.concatenate(
            [och[:, h * Dh:(h + 1) * Dh] * linv[:, h:h + 1]
             for h in range(Hq)],
            axis=1,
        )
        out_ref[orows, :] = jnp.dot(
            norm.astype(BF), wo_ref[:, :].astype(BF),
            preferred_element_type=jnp.float32,
        )

        sends = []
        if not SKIP_COMM:
            for d in range(1, N_DEV):
                dest = lax.rem(my + d, N_DEV)
                rd = pltpu.make_async_remote_copy(
                    src_ref=out_ref.at[orows, :],
                    dst_ref=out_ref.at[orows, :],
                    send_sem=ag_ssems.at[d],
                    recv_sem=ag_rsems.at[N_DEV - d],
                    device_id=(dest,), device_id_type=pl.DeviceIdType.MESH,
                )
                rd.start()
                sends.append(rd)
            for e in range(1, N_DEV):
                rc = lax.rem(my + e, N_DEV)
                rrows = pl.ds(rc * CHUNK, CHUNK)
                rd = pltpu.make_async_remote_copy(
                    src_ref=out_ref.at[rrows, :],
                    dst_ref=out_ref.at[rrows, :],
                    send_sem=ag_ssems.at[e],
                    recv_sem=ag_rsems.at[e],
                    device_id=(right,), device_id_type=pl.DeviceIdType.MESH,
                )
                rd.wait_recv()
            for rd in sends:
                rd.wait_send()

    out = pl.pallas_call(
        body,
        out_shape=jax.ShapeDtypeStruct((R, D), jnp.float32),
        in_specs=[pl.BlockSpec(memory_space=pltpu.VMEM)] * 5,
        out_specs=pl.BlockSpec(memory_space=pltpu.VMEM),
        scratch_shapes=[
            pltpu.VMEM((R, W), jnp.float32),
            pltpu.VMEM((N_DEV, CHUNK, W), jnp.float32),
            pltpu.SemaphoreType.DMA((N_DEV,)),
            pltpu.SemaphoreType.DMA((N_DEV,)),
            pltpu.SemaphoreType.DMA((N_DEV,)),
            pltpu.SemaphoreType.DMA((N_DEV,)),
        ],
        compiler_params=pltpu.CompilerParams(
            collective_id=0,
            vmem_limit_bytes=100 * 1024 * 1024,
        ),
    )(x_flat, Wq, Wo, K_ext, V_ext)
    return out.reshape(B, Sq, D)


# device time: 96958 ns/iter; 1.8341x vs baseline; 1.5917x over previous
import os
SKIP_COMM = bool(int(os.environ.get('SKIP_COMM', '0')))
import jax
import jax.numpy as jnp
from jax import lax
from jax.experimental import pallas as pl
from jax.experimental.pallas import tpu as pltpu

N_DEV = 32
B, Sq, Hq, Hkv, Dh = 4, 256, 8, 2, 128
G = Hq // Hkv
D = Hq * Dh
W = D + 128
R = B * Sq
CHUNK = R // N_DEV
SCALE = 0.08838834764831843
NR = N_DEV // 2
NL = N_DEV // 2 - 1
BF = jnp.bfloat16


def kernel(x, Wq, Wo, K_ext, V_ext):
    x_flat = x.reshape(R, D)

    def body(x_ref, wq_ref, wo_ref, k_ref, v_ref, out_ref,
             acc, acc16, out16, rs_land,
             rs_ssems, rs_rsems, ag_ssems, ag_rsems):
        my = lax.axis_index("i")
        left = lax.rem(my - 1 + N_DEV, N_DEV)
        right = lax.rem(my + 1, N_DEV)

        barrier = pltpu.get_barrier_semaphore()
        for nbr in (left, right):
            pl.semaphore_signal(
                barrier, inc=1,
                device_id=(nbr,), device_id_type=pl.DeviceIdType.MESH,
            )
        pl.semaphore_wait(barrier, 2)

        out_ref[:, :] = jnp.dot(
            x_ref[:, :].astype(BF), wq_ref[:, :].astype(BF),
            preferred_element_type=jnp.float32,
        )
        for b in range(B):
            rows = slice(b * Sq, (b + 1) * Sq)
            for g in range(Hkv):
                k16 = k_ref[b, :, g, :].astype(BF)
                v16 = v_ref[b, :, g, :].astype(BF)
                for h in range(g * G, (g + 1) * G):
                    q16 = out_ref[rows, h * Dh:(h + 1) * Dh].astype(BF)
                    s = lax.dot_general(
                        q16, k16, (((1,), (1,)), ((), ())),
                        preferred_element_type=jnp.float32,
                    ) * SCALE
                    p = jnp.exp(s)
                    l_loc = jnp.sum(p, axis=1, keepdims=True)
                    o = jnp.dot(p.astype(BF), v16,
                                preferred_element_type=jnp.float32)
                    acc[rows, h * Dh:(h + 1) * Dh] = o
                    acc[rows, D + h:D + h + 1] = l_loc

        orows = pl.ds(my * CHUNK, CHUNK)
        acc16[:, :] = acc[:, :].astype(BF)
        rs_sends = []
        if not SKIP_COMM:
            for d in range(1, N_DEV):
                dest = lax.rem(my + d, N_DEV)
                drows = pl.ds(dest * CHUNK, CHUNK)
                rd = pltpu.make_async_remote_copy(
                    src_ref=acc16.at[drows, :],
                    dst_ref=rs_land.at[N_DEV - d],
                    send_sem=rs_ssems.at[d],
                    recv_sem=rs_rsems.at[N_DEV - d],
                    device_id=(dest,), device_id_type=pl.DeviceIdType.MESH,
                )
                rd.start()
                rs_sends.append(rd)
            for e in range(1, N_DEV):
                rd = pltpu.make_async_remote_copy(
                    src_ref=acc16.at[orows, :],
                    dst_ref=rs_land.at[e],
                    send_sem=rs_ssems.at[e],
                    recv_sem=rs_rsems.at[e],
                    device_id=(right,), device_id_type=pl.DeviceIdType.MESH,
                )
                rd.wait_recv()
                acc[orows, :] = acc[orows, :] + rs_land[e].astype(jnp.float32)
            for rd in rs_sends:
                rd.wait_send()

        linv = 1.0 / acc[orows, D:D + Hq]
        och = acc[orows, 0:D]
        norm = jnp.concatenate(
            [och[:, h * Dh:(h + 1) * Dh] * linv[:, h:h + 1]
             for h in range(Hq)],
            axis=1,
        )
        out16[orows, :] = jnp.dot(
            norm.astype(BF), wo_ref[:, :].astype(BF),
            preferred_element_type=jnp.float32,
        ).astype(BF)

        sends = []
        if not SKIP_COMM:
            for d in range(1, N_DEV):
                dest = lax.rem(my + d, N_DEV)
                rd = pltpu.make_async_remote_copy(
                    src_ref=out16.at[orows, :],
                    dst_ref=out16.at[orows, :],
                    send_sem=ag_ssems.at[d],
                    recv_sem=ag_rsems.at[N_DEV - d],
                    device_id=(dest,), device_id_type=pl.DeviceIdType.MESH,
                )
                rd.start()
                sends.append(rd)
            for e in range(1, N_DEV):
                rc = lax.rem(my + e, N_DEV)
                rrows = pl.ds(rc * CHUNK, CHUNK)
                rd = pltpu.make_async_remote_copy(
                    src_ref=out16.at[rrows, :],
                    dst_ref=out16.at[rrows, :],
                    send_sem=ag_ssems.at[e],
                    recv_sem=ag_rsems.at[e],
                    device_id=(right,), device_id_type=pl.DeviceIdType.MESH,
                )
                rd.wait_recv()
            for rd in sends:
                rd.wait_send()
        out_ref[:, :] = out16[:, :].astype(jnp.float32)

    out = pl.pallas_call(
        body,
        out_shape=jax.ShapeDtypeStruct((R, D), jnp.float32),
        in_specs=[pl.BlockSpec(memory_space=pltpu.VMEM)] * 5,
        out_specs=pl.BlockSpec(memory_space=pltpu.VMEM),
        scratch_shapes=[
            pltpu.VMEM((R, W), jnp.float32),
            pltpu.VMEM((R, W), BF),
            pltpu.VMEM((R, D), BF),
            pltpu.VMEM((N_DEV, CHUNK, W), BF),
            pltpu.SemaphoreType.DMA((N_DEV,)),
            pltpu.SemaphoreType.DMA((N_DEV,)),
            pltpu.SemaphoreType.DMA((N_DEV,)),
            pltpu.SemaphoreType.DMA((N_DEV,)),
        ],
        compiler_params=pltpu.CompilerParams(
            collective_id=0,
            vmem_limit_bytes=100 * 1024 * 1024,
        ),
    )(x_flat, Wq, Wo, K_ext, V_ext)
    return out.reshape(B, Sq, D)
